# baseline (speedup 1.0000x reference)
import numpy as np
import jax
import jax.numpy as jnp
from jax.experimental import pallas as pl
from jax.experimental.pallas import tpu as pltpu

_MODEL_DIM = 1024
_SEQ_LEN = 4096
_NUM_ROWS = 4
_SPAN_LEN = 10
_NUM_SPANS = 266
_NOISE_STD = 0.1

_BS = 512
_NSB = _SEQ_LEN // _BS


def _threefry_noise_bits(cnt):
    u32 = jnp.uint32
    ks0 = u32(0)
    ks1 = u32(2)
    ks2 = ks0 ^ ks1 ^ u32(0x1BD11BDA)

    def rnds(x0, x1, rots):
        for r in rots:
            x0 = x0 + x1
            x1 = ((x1 << u32(r)) | (x1 >> u32(32 - r))) ^ x0
        return x0, x1

    x0 = jnp.zeros_like(cnt) + ks0
    x1 = cnt + ks1
    x0, x1 = rnds(x0, x1, (13, 15, 26, 6))
    x0, x1 = x0 + ks1, x1 + ks2 + u32(1)
    x0, x1 = rnds(x0, x1, (17, 29, 16, 24))
    x0, x1 = x0 + ks2, x1 + ks0 + u32(2)
    x0, x1 = rnds(x0, x1, (13, 15, 26, 6))
    x0, x1 = x0 + ks0, x1 + ks1 + u32(3)
    x0, x1 = rnds(x0, x1, (17, 29, 16, 24))
    x0, x1 = x0 + ks1, x1 + ks2 + u32(4)
    x0, x1 = rnds(x0, x1, (13, 15, 26, 6))
    x0, x1 = x0 + ks2, x1 + ks0 + u32(5)
    return x0 ^ x1


def _erfinv(x):
    f32 = jnp.float32
    w = -jnp.log((f32(1.0) - x) * (f32(1.0) + x))
    ws = w - f32(2.5)
    ps = f32(2.81022636e-08)
    for c in (3.43273939e-07, -3.5233877e-06, -4.39150654e-06, 0.00021858087,
              -0.00125372503, -0.00417768164, 0.246640727, 1.50140941):
        ps = f32(c) + ps * ws
    wb = jnp.sqrt(w) - f32(3.0)
    pb = f32(-0.000200214257)
    for c in (0.000100950558, 0.00134934322, -0.00367342844, 0.00573950773,
              -0.0076224613, 0.00943887047, 1.00167406, 2.83297682):
        pb = f32(c) + pb * wb
    return jnp.where(w < f32(5.0), ps, pb) * x


def _mask_noise_body(mask_ref, seqs_ref, out_ref):
    n = pl.program_id(0)
    sb = pl.program_id(1)
    row0 = (n * _SEQ_LEN + sb * _BS) * _MODEL_DIM
    s_io = jax.lax.broadcasted_iota(jnp.int32, (_BS, _MODEL_DIM), 0)
    m_io = jax.lax.broadcasted_iota(jnp.int32, (_BS, _MODEL_DIM), 1)
    cnt = (row0 + s_io * _MODEL_DIM + m_io).astype(jnp.uint32)
    bits = _threefry_noise_bits(cnt)
    u01 = jax.lax.bitcast_convert_type(
        (bits >> jnp.uint32(9)) | jnp.uint32(0x3F800000), jnp.float32
    ) - jnp.float32(1.0)
    lo = jnp.float32(np.nextafter(np.float32(-1.0), np.float32(0.0)))
    hi = jnp.float32(1.0)
    u = jnp.maximum(lo, u01 * (hi - lo) + lo)
    noise = jnp.float32(_NOISE_STD * np.sqrt(2.0)) * _erfinv(u)
    msk = mask_ref[0] != 0
    out_ref[0] = jnp.where(msk, noise, seqs_ref[0])


def _apply_mask_noise(mask_i32, seqs):
    return pl.pallas_call(
        _mask_noise_body,
        grid=(_NUM_ROWS, _NSB),
        in_specs=[
            pl.BlockSpec((1, _BS, 1), lambda n, sb: (n, sb, 0)),
            pl.BlockSpec((1, _BS, _MODEL_DIM), lambda n, sb: (n, sb, 0)),
        ],
        out_specs=pl.BlockSpec((1, _BS, _MODEL_DIM), lambda n, sb: (n, sb, 0)),
        out_shape=jax.ShapeDtypeStruct(seqs.shape, seqs.dtype),
        compiler_params=pltpu.CompilerParams(
            dimension_semantics=("parallel", "parallel"),
        ),
    )(mask_i32, seqs)


def kernel(seqs):
    starts = jax.random.randint(
        jax.random.key(1), (_NUM_ROWS, _NUM_SPANS), 0, _SEQ_LEN - _SPAN_LEN + 1
    )
    offsets = jnp.arange(_SPAN_LEN, dtype=starts.dtype)
    idx = starts[:, :, None] + offsets[None, None, :]
    rows = jnp.broadcast_to(
        jnp.arange(_NUM_ROWS, dtype=idx.dtype)[:, None, None], idx.shape
    )
    mask = jnp.zeros((_NUM_ROWS, _SEQ_LEN), dtype=jnp.bool_)
    mask = mask.at[rows.reshape(-1), idx.reshape(-1)].set(True)
    mask_i32 = mask.astype(jnp.int32).reshape(_NUM_ROWS, _SEQ_LEN, 1)
    out = _apply_mask_noise(mask_i32, seqs)
    return out, mask


# speedup vs baseline: 1.1233x; 1.1233x over previous
import numpy as np
import jax
import jax.numpy as jnp
from jax.experimental import pallas as pl
from jax.experimental.pallas import tpu as pltpu

_MODEL_DIM = 1024
_SEQ_LEN = 4096
_NUM_ROWS = 4
_SPAN_LEN = 10
_NUM_SPANS = 266
_NOISE_STD = 0.1

_BS = 512
_NSB = _SEQ_LEN // _BS


def _threefry_noise_bits(cnt):
    u32 = jnp.uint32
    ks0 = u32(0)
    ks1 = u32(2)
    ks2 = ks0 ^ ks1 ^ u32(0x1BD11BDA)

    def rnds(x0, x1, rots):
        for r in rots:
            x0 = x0 + x1
            x1 = ((x1 << u32(r)) | (x1 >> u32(32 - r))) ^ x0
        return x0, x1

    x0 = jnp.zeros_like(cnt) + ks0
    x1 = cnt + ks1
    x0, x1 = rnds(x0, x1, (13, 15, 26, 6))
    x0, x1 = x0 + ks1, x1 + ks2 + u32(1)
    x0, x1 = rnds(x0, x1, (17, 29, 16, 24))
    x0, x1 = x0 + ks2, x1 + ks0 + u32(2)
    x0, x1 = rnds(x0, x1, (13, 15, 26, 6))
    x0, x1 = x0 + ks0, x1 + ks1 + u32(3)
    x0, x1 = rnds(x0, x1, (17, 29, 16, 24))
    x0, x1 = x0 + ks1, x1 + ks2 + u32(4)
    x0, x1 = rnds(x0, x1, (13, 15, 26, 6))
    x0, x1 = x0 + ks2, x1 + ks0 + u32(5)
    return x0 ^ x1


def _erfinv(x):
    f32 = jnp.float32
    w = -jnp.log((f32(1.0) - x) * (f32(1.0) + x))
    ws = w - f32(2.5)
    ps = f32(2.81022636e-08)
    for c in (3.43273939e-07, -3.5233877e-06, -4.39150654e-06, 0.00021858087,
              -0.00125372503, -0.00417768164, 0.246640727, 1.50140941):
        ps = f32(c) + ps * ws
    wb = jnp.sqrt(w) - f32(3.0)
    pb = f32(-0.000200214257)
    for c in (0.000100950558, 0.00134934322, -0.00367342844, 0.00573950773,
              -0.0076224613, 0.00943887047, 1.00167406, 2.83297682):
        pb = f32(c) + pb * wb
    return jnp.where(w < f32(5.0), ps, pb) * x


def _mask_noise_body(mask_ref, seqs_ref, out_ref):
    n = pl.program_id(0)
    sb = pl.program_id(1)
    row0 = (n * _SEQ_LEN + sb * _BS) * _MODEL_DIM
    s_io = jax.lax.broadcasted_iota(jnp.int32, (_BS, _MODEL_DIM), 0)
    m_io = jax.lax.broadcasted_iota(jnp.int32, (_BS, _MODEL_DIM), 1)
    cnt = (row0 + s_io * _MODEL_DIM + m_io).astype(jnp.uint32)
    bits = _threefry_noise_bits(cnt)
    u01 = jax.lax.bitcast_convert_type(
        (bits >> jnp.uint32(9)) | jnp.uint32(0x3F800000), jnp.float32
    ) - jnp.float32(1.0)
    lo = jnp.float32(np.nextafter(np.float32(-1.0), np.float32(0.0)))
    hi = jnp.float32(1.0)
    u = jnp.maximum(lo, u01 * (hi - lo) + lo)
    noise = jnp.float32(_NOISE_STD * np.sqrt(2.0)) * _erfinv(u)
    msk = mask_ref[0] != 0
    out_ref[0] = jnp.where(msk, noise, seqs_ref[0])


def _apply_mask_noise(mask_i32, seqs):
    return pl.pallas_call(
        _mask_noise_body,
        grid=(_NUM_ROWS, _NSB),
        in_specs=[
            pl.BlockSpec((1, _BS, 1), lambda n, sb: (n, sb, 0)),
            pl.BlockSpec((1, _BS, _MODEL_DIM), lambda n, sb: (n, sb, 0)),
        ],
        out_specs=pl.BlockSpec((1, _BS, _MODEL_DIM), lambda n, sb: (n, sb, 0)),
        out_shape=jax.ShapeDtypeStruct(seqs.shape, seqs.dtype),
        compiler_params=pltpu.CompilerParams(
            dimension_semantics=("parallel", "parallel"),
        ),
    )(mask_i32, seqs)


def kernel(seqs):
    starts = jax.random.randint(
        jax.random.key(1), (_NUM_ROWS, _NUM_SPANS), 0, _SEQ_LEN - _SPAN_LEN + 1
    )
    s_iota = jnp.arange(_SEQ_LEN, dtype=starts.dtype)
    ge = s_iota[None, None, :] >= starts[:, :, None]
    lt = s_iota[None, None, :] < (starts + _SPAN_LEN)[:, :, None]
    mask = jnp.any(ge & lt, axis=1)
    mask_i32 = mask.astype(jnp.int32).reshape(_NUM_ROWS, _SEQ_LEN, 1)
    out = _apply_mask_noise(mask_i32, seqs)
    return out, mask


# speedup vs baseline: 2.0069x; 1.7866x over previous
import numpy as np
import jax
import jax.numpy as jnp
from jax.experimental import pallas as pl
from jax.experimental.pallas import tpu as pltpu

_MODEL_DIM = 1024
_SEQ_LEN = 4096
_NUM_ROWS = 4
_SPAN_LEN = 10
_NUM_SPANS = 266
_NOISE_STD = 0.1

_BS = 512
_NSB = _SEQ_LEN // _BS


def _threefry_noise_bits(cnt):
    u32 = jnp.uint32
    ks0 = u32(0)
    ks1 = u32(2)
    ks2 = ks0 ^ ks1 ^ u32(0x1BD11BDA)

    def rnds(x0, x1, rots):
        for r in rots:
            x0 = x0 + x1
            x1 = ((x1 << u32(r)) | (x1 >> u32(32 - r))) ^ x0
        return x0, x1

    x0 = jnp.zeros_like(cnt) + ks0
    x1 = cnt + ks1
    x0, x1 = rnds(x0, x1, (13, 15, 26, 6))
    x0, x1 = x0 + ks1, x1 + ks2 + u32(1)
    x0, x1 = rnds(x0, x1, (17, 29, 16, 24))
    x0, x1 = x0 + ks2, x1 + ks0 + u32(2)
    x0, x1 = rnds(x0, x1, (13, 15, 26, 6))
    x0, x1 = x0 + ks0, x1 + ks1 + u32(3)
    x0, x1 = rnds(x0, x1, (17, 29, 16, 24))
    x0, x1 = x0 + ks1, x1 + ks2 + u32(4)
    x0, x1 = rnds(x0, x1, (13, 15, 26, 6))
    x0, x1 = x0 + ks2, x1 + ks0 + u32(5)
    return x0 ^ x1


def _erfinv(x):
    f32 = jnp.float32
    w = -jnp.log((f32(1.0) - x) * (f32(1.0) + x))
    ws = w - f32(2.5)
    ps = f32(2.81022636e-08)
    for c in (3.43273939e-07, -3.5233877e-06, -4.39150654e-06, 0.00021858087,
              -0.00125372503, -0.00417768164, 0.246640727, 1.50140941):
        ps = f32(c) + ps * ws
    wb = jnp.sqrt(w) - f32(3.0)
    pb = f32(-0.000200214257)
    for c in (0.000100950558, 0.00134934322, -0.00367342844, 0.00573950773,
              -0.0076224613, 0.00943887047, 1.00167406, 2.83297682):
        pb = f32(c) + pb * wb
    return jnp.where(w < f32(5.0), ps, pb) * x


_CH = 8


def _mask_noise_body(mask_ref, seqs_ref, out_ref):
    n = pl.program_id(0)
    sb = pl.program_id(1)
    row0 = (n * _SEQ_LEN + sb * _BS) * _MODEL_DIM
    s_io = jax.lax.broadcasted_iota(jnp.int32, (_CH, _MODEL_DIM), 0)
    m_io = jax.lax.broadcasted_iota(jnp.int32, (_CH, _MODEL_DIM), 1)
    cnt0 = (row0 + s_io * _MODEL_DIM + m_io).astype(jnp.uint32)
    lo = jnp.float32(np.nextafter(np.float32(-1.0), np.float32(0.0)))
    hi = jnp.float32(1.0)

    def chunk(i, _):
        cnt = cnt0 + jnp.uint32(i * _CH * _MODEL_DIM)
        bits = _threefry_noise_bits(cnt)
        u01 = jax.lax.bitcast_convert_type(
            (bits >> jnp.uint32(9)) | jnp.uint32(0x3F800000), jnp.float32
        ) - jnp.float32(1.0)
        u = jnp.maximum(lo, u01 * (hi - lo) + lo)
        noise = jnp.float32(_NOISE_STD * np.sqrt(2.0)) * _erfinv(u)
        msk = mask_ref[0, pl.ds(i * _CH, _CH)] != 0
        out_ref[0, pl.ds(i * _CH, _CH)] = jnp.where(
            msk, noise, seqs_ref[0, pl.ds(i * _CH, _CH)]
        )
        return 0

    jax.lax.fori_loop(0, _BS // _CH, chunk, 0)


def _apply_mask_noise(mask_i32, seqs):
    return pl.pallas_call(
        _mask_noise_body,
        grid=(_NUM_ROWS, _NSB),
        in_specs=[
            pl.BlockSpec((1, _BS, 1), lambda n, sb: (n, sb, 0)),
            pl.BlockSpec((1, _BS, _MODEL_DIM), lambda n, sb: (n, sb, 0)),
        ],
        out_specs=pl.BlockSpec((1, _BS, _MODEL_DIM), lambda n, sb: (n, sb, 0)),
        out_shape=jax.ShapeDtypeStruct(seqs.shape, seqs.dtype),
        compiler_params=pltpu.CompilerParams(
            dimension_semantics=("parallel", "parallel"),
        ),
    )(mask_i32, seqs)


def kernel(seqs):
    starts = jax.random.randint(
        jax.random.key(1), (_NUM_ROWS, _NUM_SPANS), 0, _SEQ_LEN - _SPAN_LEN + 1
    )
    s_iota = jnp.arange(_SEQ_LEN, dtype=starts.dtype)
    ge = s_iota[None, None, :] >= starts[:, :, None]
    lt = s_iota[None, None, :] < (starts + _SPAN_LEN)[:, :, None]
    mask = jnp.any(ge & lt, axis=1)
    mask_i32 = mask.astype(jnp.int32).reshape(_NUM_ROWS, _SEQ_LEN, 1)
    out = _apply_mask_noise(mask_i32, seqs)
    return out, mask


# speedup vs baseline: 2.7838x; 1.3872x over previous
import numpy as np
import jax
import jax.numpy as jnp
from jax.experimental import pallas as pl
from jax.experimental.pallas import tpu as pltpu

_MODEL_DIM = 1024
_SEQ_LEN = 4096
_NUM_ROWS = 4
_SPAN_LEN = 10
_NUM_SPANS = 266
_NOISE_STD = 0.1

_BS = 512
_NSB = _SEQ_LEN // _BS
_CH = 8
_NCH = _BS // _CH

_STARTS_NP = np.asarray(
    jax.random.randint(
        jax.random.key(1), (_NUM_ROWS, _NUM_SPANS), 0, _SEQ_LEN - _SPAN_LEN + 1
    )
)

_MASK_NP = np.zeros((_NUM_ROWS, _SEQ_LEN), dtype=bool)
for _n in range(_NUM_ROWS):
    for _st in _STARTS_NP[_n]:
        _MASK_NP[_n, _st:_st + _SPAN_LEN] = True

_SCHED_NP = np.zeros((_NUM_ROWS, _NSB, _NCH), dtype=np.int32)
_NM_NP = np.zeros((_NUM_ROWS, _NSB), dtype=np.int32)
for _n in range(_NUM_ROWS):
    for _sb in range(_NSB):
        _blk = _MASK_NP[_n, _sb * _BS:(_sb + 1) * _BS].reshape(_NCH, _CH)
        _m = np.where(_blk.any(axis=1))[0]
        _u = np.where(~_blk.any(axis=1))[0]
        _SCHED_NP[_n, _sb] = np.concatenate([_m, _u]).astype(np.int32)
        _NM_NP[_n, _sb] = len(_m)


def _threefry_noise_bits(cnt):
    u32 = jnp.uint32
    ks0 = u32(0)
    ks1 = u32(2)
    ks2 = ks0 ^ ks1 ^ u32(0x1BD11BDA)

    def rnds(x0, x1, rots):
        for r in rots:
            x0 = x0 + x1
            x1 = ((x1 << u32(r)) | (x1 >> u32(32 - r))) ^ x0
        return x0, x1

    x0 = jnp.zeros_like(cnt) + ks0
    x1 = cnt + ks1
    x0, x1 = rnds(x0, x1, (13, 15, 26, 6))
    x0, x1 = x0 + ks1, x1 + ks2 + u32(1)
    x0, x1 = rnds(x0, x1, (17, 29, 16, 24))
    x0, x1 = x0 + ks2, x1 + ks0 + u32(2)
    x0, x1 = rnds(x0, x1, (13, 15, 26, 6))
    x0, x1 = x0 + ks0, x1 + ks1 + u32(3)
    x0, x1 = rnds(x0, x1, (17, 29, 16, 24))
    x0, x1 = x0 + ks1, x1 + ks2 + u32(4)
    x0, x1 = rnds(x0, x1, (13, 15, 26, 6))
    x0, x1 = x0 + ks2, x1 + ks0 + u32(5)
    return x0 ^ x1


def _erfinv(x):
    f32 = jnp.float32
    w = -jnp.log((f32(1.0) - x) * (f32(1.0) + x))
    ws = w - f32(2.5)
    ps = f32(2.81022636e-08)
    for c in (3.43273939e-07, -3.5233877e-06, -4.39150654e-06, 0.00021858087,
              -0.00125372503, -0.00417768164, 0.246640727, 1.50140941):
        ps = f32(c) + ps * ws
    wb = jnp.sqrt(w) - f32(3.0)
    pb = f32(-0.000200214257)
    for c in (0.000100950558, 0.00134934322, -0.00367342844, 0.00573950773,
              -0.0076224613, 0.00943887047, 1.00167406, 2.83297682):
        pb = f32(c) + pb * wb
    return jnp.where(w < f32(5.0), ps, pb) * x


def _mask_noise_body(sched_ref, nm_ref, mask_ref, seqs_ref, out_ref):
    n = pl.program_id(0)
    sb = pl.program_id(1)
    row0 = (n * _SEQ_LEN + sb * _BS) * _MODEL_DIM
    s_io = jax.lax.broadcasted_iota(jnp.int32, (_CH, _MODEL_DIM), 0)
    m_io = jax.lax.broadcasted_iota(jnp.int32, (_CH, _MODEL_DIM), 1)
    cnt0 = (row0 + s_io * _MODEL_DIM + m_io).astype(jnp.uint32)
    lo = jnp.float32(np.nextafter(np.float32(-1.0), np.float32(0.0)))
    hi = jnp.float32(1.0)
    nm = nm_ref[n, sb]

    def compute_chunk(k, _):
        cid = sched_ref[n, sb, k]
        base = cid * _CH
        cnt = cnt0 + (base * _MODEL_DIM).astype(jnp.uint32)
        bits = _threefry_noise_bits(cnt)
        u01 = jax.lax.bitcast_convert_type(
            (bits >> jnp.uint32(9)) | jnp.uint32(0x3F800000), jnp.float32
        ) - jnp.float32(1.0)
        u = jnp.maximum(lo, u01 * (hi - lo) + lo)
        noise = jnp.float32(_NOISE_STD * np.sqrt(2.0)) * _erfinv(u)
        msk = mask_ref[0, pl.ds(base, _CH)] != 0
        out_ref[0, pl.ds(base, _CH)] = jnp.where(
            msk, noise, seqs_ref[0, pl.ds(base, _CH)]
        )
        return 0

    def copy_chunk(k, _):
        cid = sched_ref[n, sb, k]
        base = cid * _CH
        out_ref[0, pl.ds(base, _CH)] = seqs_ref[0, pl.ds(base, _CH)]
        return 0

    jax.lax.fori_loop(0, nm, compute_chunk, 0)
    jax.lax.fori_loop(nm, _NCH, copy_chunk, 0)


def _apply_mask_noise(mask_i32, seqs):
    return pl.pallas_call(
        _mask_noise_body,
        grid_spec=pltpu.PrefetchScalarGridSpec(
            num_scalar_prefetch=2,
            grid=(_NUM_ROWS, _NSB),
            in_specs=[
                pl.BlockSpec((1, _BS, 1), lambda n, sb, *_: (n, sb, 0)),
                pl.BlockSpec((1, _BS, _MODEL_DIM), lambda n, sb, *_: (n, sb, 0)),
            ],
            out_specs=pl.BlockSpec(
                (1, _BS, _MODEL_DIM), lambda n, sb, *_: (n, sb, 0)
            ),
        ),
        out_shape=jax.ShapeDtypeStruct(seqs.shape, seqs.dtype),
        compiler_params=pltpu.CompilerParams(
            dimension_semantics=("parallel", "parallel"),
        ),
    )(jnp.asarray(_SCHED_NP), jnp.asarray(_NM_NP), mask_i32, seqs)


def kernel(seqs):
    starts = jax.random.randint(
        jax.random.key(1), (_NUM_ROWS, _NUM_SPANS), 0, _SEQ_LEN - _SPAN_LEN + 1
    )
    s_iota = jnp.arange(_SEQ_LEN, dtype=starts.dtype)
    ge = s_iota[None, None, :] >= starts[:, :, None]
    lt = s_iota[None, None, :] < (starts + _SPAN_LEN)[:, :, None]
    mask = jnp.any(ge & lt, axis=1)
    mask_i32 = mask.astype(jnp.int32).reshape(_NUM_ROWS, _SEQ_LEN, 1)
    out = _apply_mask_noise(mask_i32, seqs)
    return out, mask


# speedup vs baseline: 2.8316x; 1.0171x over previous
import numpy as np
import jax
import jax.numpy as jnp
from jax.experimental import pallas as pl
from jax.experimental.pallas import tpu as pltpu

_MODEL_DIM = 1024
_SEQ_LEN = 4096
_NUM_ROWS = 4
_SPAN_LEN = 10
_NUM_SPANS = 266
_NOISE_STD = 0.1

_BS = 512
_NSB = _SEQ_LEN // _BS
_CH = 8
_NCH = _BS // _CH

_STARTS_NP = np.asarray(
    jax.random.randint(
        jax.random.key(1), (_NUM_ROWS, _NUM_SPANS), 0, _SEQ_LEN - _SPAN_LEN + 1
    )
)

_MASK_NP = np.zeros((_NUM_ROWS, _SEQ_LEN), dtype=bool)
for _n in range(_NUM_ROWS):
    for _st in _STARTS_NP[_n]:
        _MASK_NP[_n, _st:_st + _SPAN_LEN] = True

_SCHED_M_NP = np.zeros((_NUM_ROWS, _NSB, _NCH + 1), dtype=np.int32)
_SCHED_U_NP = np.zeros((_NUM_ROWS, _NSB, _NCH), dtype=np.int32)
_NM_NP = np.zeros((_NUM_ROWS, _NSB), dtype=np.int32)
_NU_NP = np.zeros((_NUM_ROWS, _NSB), dtype=np.int32)
for _n in range(_NUM_ROWS):
    for _sb in range(_NSB):
        _blk = _MASK_NP[_n, _sb * _BS:(_sb + 1) * _BS].reshape(_NCH, _CH)
        _m = list(np.where(_blk.any(axis=1))[0])
        _u = list(np.where(~_blk.any(axis=1))[0])
        if len(_m) % 2:
            _m.append(_m[-1])
        _SCHED_M_NP[_n, _sb, :len(_m)] = _m
        _SCHED_U_NP[_n, _sb, :len(_u)] = _u
        _NM_NP[_n, _sb] = len(_m) // 2
        _NU_NP[_n, _sb] = len(_u)


def _threefry_noise_bits(cnt):
    u32 = jnp.uint32
    ks0 = u32(0)
    ks1 = u32(2)
    ks2 = ks0 ^ ks1 ^ u32(0x1BD11BDA)

    def rnds(x0, x1, rots):
        for r in rots:
            x0 = x0 + x1
            x1 = ((x1 << u32(r)) | (x1 >> u32(32 - r))) ^ x0
        return x0, x1

    x0 = jnp.zeros_like(cnt) + ks0
    x1 = cnt + ks1
    x0, x1 = rnds(x0, x1, (13, 15, 26, 6))
    x0, x1 = x0 + ks1, x1 + ks2 + u32(1)
    x0, x1 = rnds(x0, x1, (17, 29, 16, 24))
    x0, x1 = x0 + ks2, x1 + ks0 + u32(2)
    x0, x1 = rnds(x0, x1, (13, 15, 26, 6))
    x0, x1 = x0 + ks0, x1 + ks1 + u32(3)
    x0, x1 = rnds(x0, x1, (17, 29, 16, 24))
    x0, x1 = x0 + ks1, x1 + ks2 + u32(4)
    x0, x1 = rnds(x0, x1, (13, 15, 26, 6))
    x0, x1 = x0 + ks2, x1 + ks0 + u32(5)
    return x0 ^ x1


def _erfinv(x):
    f32 = jnp.float32
    w = -jnp.log((f32(1.0) - x) * (f32(1.0) + x))
    ws = w - f32(2.5)
    ps = f32(2.81022636e-08)
    for c in (3.43273939e-07, -3.5233877e-06, -4.39150654e-06, 0.00021858087,
              -0.00125372503, -0.00417768164, 0.246640727, 1.50140941):
        ps = f32(c) + ps * ws
    wb = jnp.sqrt(w) - f32(3.0)
    pb = f32(-0.000200214257)
    for c in (0.000100950558, 0.00134934322, -0.00367342844, 0.00573950773,
              -0.0076224613, 0.00943887047, 1.00167406, 2.83297682):
        pb = f32(c) + pb * wb
    return jnp.where(w < f32(5.0), ps, pb) * x


def _mask_noise_body(schedm_ref, schedu_ref, nm_ref, nu_ref, mask_ref,
                     seqs_ref, out_ref):
    n = pl.program_id(0)
    sb = pl.program_id(1)
    row0 = (n * _SEQ_LEN + sb * _BS) * _MODEL_DIM
    s_io = jax.lax.broadcasted_iota(jnp.int32, (_CH, _MODEL_DIM), 0)
    m_io = jax.lax.broadcasted_iota(jnp.int32, (_CH, _MODEL_DIM), 1)
    cnt0 = (row0 + s_io * _MODEL_DIM + m_io).astype(jnp.uint32)
    lo = jnp.float32(np.nextafter(np.float32(-1.0), np.float32(0.0)))
    hi = jnp.float32(1.0)

    def noise_select(base):
        cnt = cnt0 + (base * _MODEL_DIM).astype(jnp.uint32)
        bits = _threefry_noise_bits(cnt)
        u01 = jax.lax.bitcast_convert_type(
            (bits >> jnp.uint32(9)) | jnp.uint32(0x3F800000), jnp.float32
        ) - jnp.float32(1.0)
        u = jnp.maximum(lo, u01 * (hi - lo) + lo)
        noise = jnp.float32(_NOISE_STD * np.sqrt(2.0)) * _erfinv(u)
        msk = mask_ref[0, pl.ds(base, _CH)] != 0
        out_ref[0, pl.ds(base, _CH)] = jnp.where(
            msk, noise, seqs_ref[0, pl.ds(base, _CH)]
        )

    def compute_pair(k, _):
        noise_select(schedm_ref[n, sb, 2 * k] * _CH)
        noise_select(schedm_ref[n, sb, 2 * k + 1] * _CH)
        return 0

    def copy_chunk(k, _):
        base = schedu_ref[n, sb, k] * _CH
        out_ref[0, pl.ds(base, _CH)] = seqs_ref[0, pl.ds(base, _CH)]
        return 0

    jax.lax.fori_loop(0, nm_ref[n, sb], compute_pair, 0)
    jax.lax.fori_loop(0, nu_ref[n, sb], copy_chunk, 0)


def _apply_mask_noise(mask_i32, seqs):
    return pl.pallas_call(
        _mask_noise_body,
        grid_spec=pltpu.PrefetchScalarGridSpec(
            num_scalar_prefetch=4,
            grid=(_NUM_ROWS, _NSB),
            in_specs=[
                pl.BlockSpec((1, _BS, 1), lambda n, sb, *_: (n, sb, 0)),
                pl.BlockSpec((1, _BS, _MODEL_DIM), lambda n, sb, *_: (n, sb, 0)),
            ],
            out_specs=pl.BlockSpec(
                (1, _BS, _MODEL_DIM), lambda n, sb, *_: (n, sb, 0)
            ),
        ),
        out_shape=jax.ShapeDtypeStruct(seqs.shape, seqs.dtype),
        compiler_params=pltpu.CompilerParams(
            dimension_semantics=("parallel", "parallel"),
        ),
    )(jnp.asarray(_SCHED_M_NP), jnp.asarray(_SCHED_U_NP),
      jnp.asarray(_NM_NP), jnp.asarray(_NU_NP), mask_i32, seqs)


def kernel(seqs):
    starts = jax.random.randint(
        jax.random.key(1), (_NUM_ROWS, _NUM_SPANS), 0, _SEQ_LEN - _SPAN_LEN + 1
    )
    s_iota = jnp.arange(_SEQ_LEN, dtype=starts.dtype)
    ge = s_iota[None, None, :] >= starts[:, :, None]
    lt = s_iota[None, None, :] < (starts + _SPAN_LEN)[:, :, None]
    mask = jnp.any(ge & lt, axis=1)
    mask_i32 = mask.astype(jnp.int32).reshape(_NUM_ROWS, _SEQ_LEN, 1)
    out = _apply_mask_noise(mask_i32, seqs)
    return out, mask
